# baseline (speedup 1.0000x reference)
import functools

import jax
import jax.numpy as jnp
import numpy as np
from jax.experimental import pallas as pl

B, N, KC, T, D, DH, P = 128, 64, 2, 32, 64, 64, 8
D2 = D // 2
TWO_PI = 2.0 * np.pi


def _fused_kernel(x_ego_ref, x_nei_ref, f_ego_ref, f_nei_ref,
                  W1_ref, b1_ref, W2_ref, b2_ref, W3_ref, b3_ref,
                  Wce_ref, bce_ref, out_ref):
    fe = f_ego_ref[0]
    fn = f_nei_ref[0]
    f = jnp.max(fn * fe[None], axis=1)

    h = f.reshape(N * T, D)
    h = jax.nn.relu(jnp.dot(h, W1_ref[...], preferred_element_type=jnp.float32)
                    + b1_ref[...])
    h = jax.nn.relu(jnp.dot(h, W2_ref[...], preferred_element_type=jnp.float32)
                    + b2_ref[...])
    f2 = jax.nn.relu(jnp.dot(h, W3_ref[...], preferred_element_type=jnp.float32)
                     + b3_ref[...])
    f2 = f2.reshape(N, T, D2)

    pn = x_nei_ref[0] - x_ego_ref[0][None]
    px = pn[..., 0]
    py = pn[..., 1]
    dist = jnp.sqrt(px * px + py * py)
    ang = jnp.mod(jnp.arctan2(px, py), TWO_PI)
    pidx = (ang / (TWO_PI / P)).astype(jnp.int32)
    valid = ((jnp.abs(px) + jnp.abs(py)) != 0) & (dist > 0.005)
    pidx = jnp.where(valid, pidx, -1)

    pe_rows = []
    ys_rows = []
    for p in range(P):
        m = (pidx == p).astype(jnp.float32)
        n = jnp.sum(m, axis=0) + 0.0001
        pd = jnp.sum(dist * m, axis=0) / n
        pa = jnp.sum(ang * m, axis=0) / n
        ys_p = jnp.sum(f2 * m[:, :, None], axis=0) / n[:, None]
        pe_p = jax.nn.relu(pd[:, None] * Wce_ref[0, :][None, :]
                           + pa[:, None] * Wce_ref[1, :][None, :]
                           + bce_ref[...])
        ys_rows.append(ys_p)
        pe_rows.append(pe_p)

    ys = jnp.stack(ys_rows, axis=1)
    pe = jnp.stack(pe_rows, axis=1)
    out_ref[0] = jnp.concatenate([ys, pe], axis=-1)


@jax.jit
def kernel(x_ego_mean, x_nei_mean, f_ego, f_nei, W1, b1, W2, b2, W3, b3,
           Wce, bce):
    b1 = b1.reshape(1, DH)
    b2 = b2.reshape(1, DH)
    b3 = b3.reshape(1, D2)
    bce = bce.reshape(1, D2)

    full = lambda shape: pl.BlockSpec(shape, lambda i: (0,) * len(shape))
    grid = (B,)
    out = pl.pallas_call(
        _fused_kernel,
        grid=grid,
        in_specs=[
            pl.BlockSpec((1, T, 2), lambda i: (i, 0, 0)),
            pl.BlockSpec((1, N, T, 2), lambda i: (i, 0, 0, 0)),
            pl.BlockSpec((1, KC, T, D), lambda i: (i, 0, 0, 0)),
            pl.BlockSpec((1, N, KC, T, D), lambda i: (i, 0, 0, 0, 0)),
            full((D, DH)), full((1, DH)),
            full((DH, DH)), full((1, DH)),
            full((DH, D2)), full((1, D2)),
            full((2, D2)), full((1, D2)),
        ],
        out_specs=pl.BlockSpec((1, T, P, D), lambda i: (i, 0, 0, 0)),
        out_shape=jax.ShapeDtypeStruct((B, T, P, D), jnp.float32),
    )(x_ego_mean, x_nei_mean, f_ego, f_nei,
      W1, b1, W2, b2, W3, b3, Wce, bce)
    return out


# speedup vs baseline: 3.9165x; 3.9165x over previous
import jax
import jax.numpy as jnp
import numpy as np
from jax.experimental import pallas as pl

B, N, KC, T, D, DH, P = 128, 64, 2, 32, 64, 64, 8
D2 = D // 2
G = 4
NT = N * T
TP = T * P
TWO_PI = np.float32(2.0 * np.pi)
PI = np.float32(np.pi)
HALF_PI = np.float32(np.pi / 2)
QUARTER_PI = np.float32(np.pi / 4)
INV_BUCKET = np.float32(P / (2.0 * np.pi))

_TMASK = (np.arange(TP)[:, None] // P == np.arange(NT)[None, :] % T)
_TMASK = _TMASK.astype(np.float32)


def _atan2_0_2pi(y, x):
    ax = jnp.abs(x)
    ay = jnp.abs(y)
    hi = jnp.maximum(ax, ay)
    lo = jnp.minimum(ax, ay)
    q = lo / jnp.where(hi == 0.0, 1.0, hi)
    big = q > 0.4142135623730950
    w = jnp.where(big, (q - 1.0) / (q + 1.0), q)
    z = w * w
    r = (((8.05374449538e-2 * z - 1.38776856032e-1) * z
          + 1.99777106478e-1) * z - 3.33329491539e-1) * z * w + w
    r = jnp.where(big, QUARTER_PI + r, r)
    r = jnp.where(ay > ax, HALF_PI - r, r)
    r = jnp.where(x < 0.0, PI - r, r)
    r = jnp.where(y < 0.0, -r, r)
    return jnp.where(r < 0.0, r + TWO_PI, r)


def _fused_kernel(px_ref, py_ref, ex_ref, ey_ref, f_ego_ref, f_nei_ref,
                  tmask_ref,
                  W1_ref, b1_ref, W2_ref, b2_ref, W3_ref, b3_ref,
                  Wce_ref, bce_ref, out_ref):
    fn = f_nei_ref[...].reshape(G * N, KC, T, D)
    fe = jnp.broadcast_to(
        f_ego_ref[...].reshape(G, 1, KC, T, D),
        (G, N, KC, T, D)).reshape(G * N, KC, T, D)
    f = jnp.max(fn * fe, axis=1)
    h = f.reshape(G * N * T, D)
    h = jax.nn.relu(jnp.dot(h, W1_ref[...], preferred_element_type=jnp.float32)
                    + b1_ref[...])
    h = jax.nn.relu(jnp.dot(h, W2_ref[...], preferred_element_type=jnp.float32)
                    + b2_ref[...])
    f2 = jax.nn.relu(jnp.dot(h, W3_ref[...], preferred_element_type=jnp.float32)
                     + b3_ref[...])

    px = px_ref[0] - ex_ref[0]
    py = py_ref[0] - ey_ref[0]
    dist = jnp.sqrt(px * px + py * py)
    ang = _atan2_0_2pi(px, py)
    valid = ((jnp.abs(px) + jnp.abs(py)) != 0.0) & (dist > 0.005)
    pidx = jnp.where(valid, (ang * INV_BUCKET).astype(jnp.int32), -1)

    tmask = tmask_ref[...]
    p_of_row = jax.lax.broadcasted_iota(jnp.int32, (TP, 1), 0) % P
    wce0 = Wce_ref[0:1, :]
    wce1 = Wce_ref[1:2, :]
    bce = bce_ref[...]
    ones_col = jnp.ones((NT, 1), jnp.float32)
    y_rows = []
    for g in range(G):
        m2 = jnp.where(pidx[g:g + 1, :] == p_of_row, tmask, 0.0)
        aux = jnp.concatenate([dist[g:g + 1, :], ang[g:g + 1, :]], axis=0)
        rhs = jnp.concatenate(
            [f2[g * NT:(g + 1) * NT, :], aux.T, ones_col], axis=1)
        out_g = jnp.dot(m2, rhs, preferred_element_type=jnp.float32)
        inv = 1.0 / (out_g[:, D2 + 2:D2 + 3] + 0.0001)
        ys = out_g[:, 0:D2] * inv
        pd = out_g[:, D2:D2 + 1] * inv
        pa = out_g[:, D2 + 1:D2 + 2] * inv
        pe = jax.nn.relu(pd * wce0 + pa * wce1 + bce)
        y_rows.append(jnp.concatenate([ys, pe], axis=1).reshape(T, P, D))
    out_ref[...] = jnp.stack(y_rows, axis=0)


@jax.jit
def kernel(x_ego_mean, x_nei_mean, f_ego, f_nei, W1, b1, W2, b2, W3, b3,
           Wce, bce):
    px_flat = x_nei_mean[..., 0].reshape(B // G, G, NT)
    py_flat = x_nei_mean[..., 1].reshape(B // G, G, NT)
    ex_t = jnp.tile(x_ego_mean[..., 0], (1, N)).reshape(B // G, G, NT)
    ey_t = jnp.tile(x_ego_mean[..., 1], (1, N)).reshape(B // G, G, NT)
    tmask = jnp.asarray(_TMASK)
    b1 = b1.reshape(1, DH)
    b2 = b2.reshape(1, DH)
    b3 = b3.reshape(1, D2)
    bce = bce.reshape(1, D2)

    full = lambda shape: pl.BlockSpec(shape, lambda i: (0,) * len(shape))
    out = pl.pallas_call(
        _fused_kernel,
        grid=(B // G,),
        in_specs=[
            pl.BlockSpec((1, G, NT), lambda i: (i, 0, 0)),
            pl.BlockSpec((1, G, NT), lambda i: (i, 0, 0)),
            pl.BlockSpec((1, G, NT), lambda i: (i, 0, 0)),
            pl.BlockSpec((1, G, NT), lambda i: (i, 0, 0)),
            pl.BlockSpec((G, KC, T, D), lambda i: (i, 0, 0, 0)),
            pl.BlockSpec((G, N, KC, T, D), lambda i: (i, 0, 0, 0, 0)),
            full((TP, NT)),
            full((D, DH)), full((1, DH)),
            full((DH, DH)), full((1, DH)),
            full((DH, D2)), full((1, D2)),
            full((2, D2)), full((1, D2)),
        ],
        out_specs=pl.BlockSpec((G, T, P, D), lambda i: (i, 0, 0, 0)),
        out_shape=jax.ShapeDtypeStruct((B, T, P, D), jnp.float32),
    )(px_flat, py_flat, ex_t, ey_t, f_ego, f_nei, tmask,
      W1, b1, W2, b2, W3, b3, Wce, bce)
    return out


# speedup vs baseline: 3.9585x; 1.0107x over previous
import jax
import jax.numpy as jnp
import numpy as np
from jax.experimental import pallas as pl

B, N, KC, T, D, DH, P = 128, 64, 2, 32, 64, 64, 8
D2 = D // 2
G = 4
NT = N * T
TP = T * P
TWO_PI = np.float32(2.0 * np.pi)
PI = np.float32(np.pi)
HALF_PI = np.float32(np.pi / 2)
QUARTER_PI = np.float32(np.pi / 4)
INV_BUCKET = np.float32(P / (2.0 * np.pi))

_TMASK = (np.arange(TP)[:, None] // P == np.arange(NT)[None, :] % T)
_TMASK = _TMASK.astype(np.float32)


def _atan2_0_2pi(y, x):
    ax = jnp.abs(x)
    ay = jnp.abs(y)
    hi = jnp.maximum(ax, ay)
    lo = jnp.minimum(ax, ay)
    q = lo / jnp.where(hi == 0.0, 1.0, hi)
    big = q > 0.4142135623730950
    w = jnp.where(big, (q - 1.0) / (q + 1.0), q)
    z = w * w
    r = (((8.05374449538e-2 * z - 1.38776856032e-1) * z
          + 1.99777106478e-1) * z - 3.33329491539e-1) * z * w + w
    r = jnp.where(big, QUARTER_PI + r, r)
    r = jnp.where(ay > ax, HALF_PI - r, r)
    r = jnp.where(x < 0.0, PI - r, r)
    r = jnp.where(y < 0.0, -r, r)
    return jnp.where(r < 0.0, r + TWO_PI, r)


def _fused_kernel(px_ref, py_ref, ex_ref, ey_ref, f_ego_ref, f_nei_ref,
                  tmask_ref,
                  W1_ref, b1_ref, W2_ref, b2_ref, W3_ref, b3_ref,
                  Wce_ref, bce_ref, out_ref):
    fn = f_nei_ref[...].reshape(G * N, KC, T, D)
    fe = jnp.broadcast_to(
        f_ego_ref[...].reshape(G, 1, KC, T, D),
        (G, N, KC, T, D)).reshape(G * N, KC, T, D)
    f = jnp.max(fn * fe, axis=1)
    h = f.reshape(G * N * T, D)
    h = jax.nn.relu(jnp.dot(h, W1_ref[...], preferred_element_type=jnp.float32)
                    + b1_ref[...])
    h = jax.nn.relu(jnp.dot(h, W2_ref[...], preferred_element_type=jnp.float32)
                    + b2_ref[...])
    f2 = jax.nn.relu(jnp.dot(h, W3_ref[...], preferred_element_type=jnp.float32)
                     + b3_ref[...])

    ex = jnp.concatenate([ex_ref[0]] * N, axis=1)
    ey = jnp.concatenate([ey_ref[0]] * N, axis=1)
    px = px_ref[0] - ex
    py = py_ref[0] - ey
    dist = jnp.sqrt(px * px + py * py)
    ang = _atan2_0_2pi(px, py)
    valid = ((jnp.abs(px) + jnp.abs(py)) != 0.0) & (dist > 0.005)
    pidx = jnp.where(valid, (ang * INV_BUCKET).astype(jnp.int32), -1)

    tmask = tmask_ref[...]
    p_of_row = jax.lax.broadcasted_iota(jnp.int32, (TP, 1), 0) % P
    wce0 = Wce_ref[0:1, :]
    wce1 = Wce_ref[1:2, :]
    bce = bce_ref[...]
    ones_col = jnp.ones((NT, 1), jnp.float32)
    y_rows = []
    for g in range(G):
        m2 = jnp.where(pidx[g:g + 1, :] == p_of_row, tmask, 0.0)
        aux = jnp.concatenate([dist[g:g + 1, :], ang[g:g + 1, :]], axis=0)
        rhs = jnp.concatenate(
            [f2[g * NT:(g + 1) * NT, :], aux.T, ones_col], axis=1)
        out_g = jnp.dot(m2, rhs, preferred_element_type=jnp.float32)
        inv = 1.0 / (out_g[:, D2 + 2:D2 + 3] + 0.0001)
        ys = out_g[:, 0:D2] * inv
        pd = out_g[:, D2:D2 + 1] * inv
        pa = out_g[:, D2 + 1:D2 + 2] * inv
        pe = jax.nn.relu(pd * wce0 + pa * wce1 + bce)
        y_rows.append(jnp.concatenate([ys, pe], axis=1).reshape(T, P, D))
    out_ref[...] = jnp.stack(y_rows, axis=0)


@jax.jit
def kernel(x_ego_mean, x_nei_mean, f_ego, f_nei, W1, b1, W2, b2, W3, b3,
           Wce, bce):
    xnt = x_nei_mean.transpose(3, 0, 1, 2)
    px_flat = xnt[0].reshape(B // G, G, NT)
    py_flat = xnt[1].reshape(B // G, G, NT)
    xet = x_ego_mean.transpose(2, 0, 1)
    ex_s = xet[0].reshape(B // G, G, T)
    ey_s = xet[1].reshape(B // G, G, T)
    tmask = jnp.asarray(_TMASK)
    b1 = b1.reshape(1, DH)
    b2 = b2.reshape(1, DH)
    b3 = b3.reshape(1, D2)
    bce = bce.reshape(1, D2)

    full = lambda shape: pl.BlockSpec(shape, lambda i: (0,) * len(shape))
    out = pl.pallas_call(
        _fused_kernel,
        grid=(B // G,),
        in_specs=[
            pl.BlockSpec((1, G, NT), lambda i: (i, 0, 0)),
            pl.BlockSpec((1, G, NT), lambda i: (i, 0, 0)),
            pl.BlockSpec((1, G, T), lambda i: (i, 0, 0)),
            pl.BlockSpec((1, G, T), lambda i: (i, 0, 0)),
            pl.BlockSpec((G, KC, T, D), lambda i: (i, 0, 0, 0)),
            pl.BlockSpec((G, N, KC, T, D), lambda i: (i, 0, 0, 0, 0)),
            full((TP, NT)),
            full((D, DH)), full((1, DH)),
            full((DH, DH)), full((1, DH)),
            full((DH, D2)), full((1, D2)),
            full((2, D2)), full((1, D2)),
        ],
        out_specs=pl.BlockSpec((G, T, P, D), lambda i: (i, 0, 0, 0)),
        out_shape=jax.ShapeDtypeStruct((B, T, P, D), jnp.float32),
    )(px_flat, py_flat, ex_s, ey_s, f_ego, f_nei, tmask,
      W1, b1, W2, b2, W3, b3, Wce, bce)
    return out


# speedup vs baseline: 4.9615x; 1.2534x over previous
import jax
import jax.numpy as jnp
import numpy as np
from jax.experimental import pallas as pl

B, N, KC, T, D, DH, P = 128, 64, 2, 32, 64, 64, 8
D2 = D // 2
G = 4
T2 = T // 2
NT2 = N * T2
TP = T * P
TWO_PI = np.float32(2.0 * np.pi)
PI = np.float32(np.pi)
HALF_PI = np.float32(np.pi / 2)
QUARTER_PI = np.float32(np.pi / 4)
INV_BUCKET = np.float32(P / (2.0 * np.pi))

_ROW_T = np.arange(TP)[:, None] // P
_COL_T2 = np.arange(NT2)[None, :] % T2
_TMASK_E = (_ROW_T == 2 * _COL_T2).astype(np.float32)
_TMASK_O = (_ROW_T == 2 * _COL_T2 + 1).astype(np.float32)


def _atan2_0_2pi(y, x):
    ax = jnp.abs(x)
    ay = jnp.abs(y)
    hi = jnp.maximum(ax, ay)
    lo = jnp.minimum(ax, ay)
    q = lo / jnp.where(hi == 0.0, 1.0, hi)
    big = q > 0.4142135623730950
    w = jnp.where(big, (q - 1.0) / (q + 1.0), q)
    z = w * w
    r = (((8.05374449538e-2 * z - 1.38776856032e-1) * z
          + 1.99777106478e-1) * z - 3.33329491539e-1) * z * w + w
    r = jnp.where(big, QUARTER_PI + r, r)
    r = jnp.where(ay > ax, HALF_PI - r, r)
    r = jnp.where(x < 0.0, PI - r, r)
    r = jnp.where(y < 0.0, -r, r)
    return jnp.where(r < 0.0, r + TWO_PI, r)


def _bucketize(px_ref, py_ref, ex_ref, ey_ref):
    ex = jnp.concatenate([ex_ref[0]] * N, axis=1)
    ey = jnp.concatenate([ey_ref[0]] * N, axis=1)
    px = px_ref[0] - ex
    py = py_ref[0] - ey
    dist = jnp.sqrt(px * px + py * py)
    ang = _atan2_0_2pi(px, py)
    valid = ((jnp.abs(px) + jnp.abs(py)) != 0.0) & (dist > 0.005)
    pidx = jnp.where(valid, (ang * INV_BUCKET).astype(jnp.int32), -1)
    return dist, ang, pidx


def _fused_kernel(pxe_ref, pye_ref, pxo_ref, pyo_ref,
                  exe_ref, eye_ref, exo_ref, eyo_ref,
                  f_ego_ref, f_nei_ref, tme_ref, tmo_ref,
                  W12_ref, b12_ref, W22_ref, b22_ref, W32_ref, b32_ref,
                  Wce_ref, bce_ref, out_ref):
    fn = f_nei_ref[...].reshape(G, N, KC, 16, 128)
    fe = jnp.broadcast_to(
        f_ego_ref[...].reshape(G, 1, KC, 16, 128),
        (G, N, KC, 16, 128))
    f = jnp.max(fn * fe, axis=2)
    h = f.reshape(G * N * 16, 128)
    h = jax.nn.relu(jnp.dot(h, W12_ref[...],
                            preferred_element_type=jnp.float32) + b12_ref[...])
    h = jax.nn.relu(jnp.dot(h, W22_ref[...],
                            preferred_element_type=jnp.float32) + b22_ref[...])
    f2 = jax.nn.relu(jnp.dot(h, W32_ref[...],
                             preferred_element_type=jnp.float32) + b32_ref[...])

    dist_e, ang_e, pidx_e = _bucketize(pxe_ref, pye_ref, exe_ref, eye_ref)
    dist_o, ang_o, pidx_o = _bucketize(pxo_ref, pyo_ref, exo_ref, eyo_ref)

    tme = tme_ref[...]
    tmo = tmo_ref[...]
    p_of_row = jax.lax.broadcasted_iota(jnp.int32, (TP, 1), 0) % P
    wce0 = Wce_ref[0:1, :]
    wce1 = Wce_ref[1:2, :]
    bce = bce_ref[...]
    ones_col = jnp.ones((NT2, 1), jnp.float32)
    y_rows = []
    for g in range(G):
        f2g = f2[g * NT2:(g + 1) * NT2, :]
        m2e = jnp.where(pidx_e[g:g + 1, :] == p_of_row, tme, 0.0)
        m2o = jnp.where(pidx_o[g:g + 1, :] == p_of_row, tmo, 0.0)
        aux_e = jnp.concatenate([dist_e[g:g + 1, :], ang_e[g:g + 1, :]], axis=0)
        aux_o = jnp.concatenate([dist_o[g:g + 1, :], ang_o[g:g + 1, :]], axis=0)
        rhs_e = jnp.concatenate([f2g[:, 0:D2], aux_e.T, ones_col], axis=1)
        rhs_o = jnp.concatenate([f2g[:, D2:D], aux_o.T, ones_col], axis=1)
        out_g = (jnp.dot(m2e, rhs_e, preferred_element_type=jnp.float32)
                 + jnp.dot(m2o, rhs_o, preferred_element_type=jnp.float32))
        inv = 1.0 / (out_g[:, D2 + 2:D2 + 3] + 0.0001)
        ys = out_g[:, 0:D2] * inv
        pd = out_g[:, D2:D2 + 1] * inv
        pa = out_g[:, D2 + 1:D2 + 2] * inv
        pe = jax.nn.relu(pd * wce0 + pa * wce1 + bce)
        y_rows.append(jnp.concatenate([ys, pe], axis=1).reshape(T, P, D))
    out_ref[...] = jnp.stack(y_rows, axis=0)


def _blockdiag(W):
    z = jnp.zeros_like(W)
    return jnp.concatenate(
        [jnp.concatenate([W, z], axis=1), jnp.concatenate([z, W], axis=1)],
        axis=0)


@jax.jit
def kernel(x_ego_mean, x_nei_mean, f_ego, f_nei, W1, b1, W2, b2, W3, b3,
           Wce, bce):
    xnt = x_nei_mean.transpose(3, 0, 1, 2)
    pxe = xnt[0][:, :, 0::2].reshape(B // G, G, NT2)
    pxo = xnt[0][:, :, 1::2].reshape(B // G, G, NT2)
    pye = xnt[1][:, :, 0::2].reshape(B // G, G, NT2)
    pyo = xnt[1][:, :, 1::2].reshape(B // G, G, NT2)
    xet = x_ego_mean.transpose(2, 0, 1)
    exe = xet[0][:, 0::2].reshape(B // G, G, T2)
    exo = xet[0][:, 1::2].reshape(B // G, G, T2)
    eye_ = xet[1][:, 0::2].reshape(B // G, G, T2)
    eyo = xet[1][:, 1::2].reshape(B // G, G, T2)
    f_nei_v = f_nei.reshape(B, N * KC * 16, 128)
    f_ego_v = f_ego.reshape(B, KC * 16, 128)
    tme = jnp.asarray(_TMASK_E)
    tmo = jnp.asarray(_TMASK_O)
    W12 = _blockdiag(W1)
    W22 = _blockdiag(W2)
    W32 = _blockdiag(W3)
    b12 = jnp.concatenate([b1, b1]).reshape(1, 2 * DH)
    b22 = jnp.concatenate([b2, b2]).reshape(1, 2 * DH)
    b32 = jnp.concatenate([b3, b3]).reshape(1, D)
    bce = bce.reshape(1, D2)

    full = lambda shape: pl.BlockSpec(shape, lambda i: (0,) * len(shape))
    out = pl.pallas_call(
        _fused_kernel,
        grid=(B // G,),
        in_specs=[
            pl.BlockSpec((1, G, NT2), lambda i: (i, 0, 0)),
            pl.BlockSpec((1, G, NT2), lambda i: (i, 0, 0)),
            pl.BlockSpec((1, G, NT2), lambda i: (i, 0, 0)),
            pl.BlockSpec((1, G, NT2), lambda i: (i, 0, 0)),
            pl.BlockSpec((1, G, T2), lambda i: (i, 0, 0)),
            pl.BlockSpec((1, G, T2), lambda i: (i, 0, 0)),
            pl.BlockSpec((1, G, T2), lambda i: (i, 0, 0)),
            pl.BlockSpec((1, G, T2), lambda i: (i, 0, 0)),
            pl.BlockSpec((G, KC * 16, 128), lambda i: (i, 0, 0)),
            pl.BlockSpec((G, N * KC * 16, 128), lambda i: (i, 0, 0)),
            full((TP, NT2)), full((TP, NT2)),
            full((2 * DH, 2 * DH)), full((1, 2 * DH)),
            full((2 * DH, 2 * DH)), full((1, 2 * DH)),
            full((2 * DH, D)), full((1, D)),
            full((2, D2)), full((1, D2)),
        ],
        out_specs=pl.BlockSpec((G, T, P, D), lambda i: (i, 0, 0, 0)),
        out_shape=jax.ShapeDtypeStruct((B, T, P, D), jnp.float32),
    )(pxe, pye, pxo, pyo, exe, eye_, exo, eyo, f_ego_v, f_nei_v, tme, tmo,
      W12, b12, W22, b22, W32, b32, Wce, bce)
    return out
